# baseline (speedup 1.0000x reference)
import jax
import jax.numpy as jnp
from jax import lax
from jax.experimental import pallas as pl
from jax.experimental.pallas import tpu as pltpu, tpu_sc as plsc

DIM = 64
BATCH = 16384

_info = plsc.get_sparse_core_info()
_NC, _NS, _L = _info.num_cores, _info.num_subcores, _info.num_lanes
_NW = _NC * _NS
_BPW = BATCH // _NW
_CH = 128
_NCH = _BPW // _CH


def _body(target_hbm, context_hbm, emb_hbm, outtab_hbm, out_hbm,
          tidx_v, cidx_v, trows_v, crows_v, res_v, sem):
    wid = lax.axis_index("s") * _NC + lax.axis_index("c")
    base = wid * _BPW

    pltpu.sync_copy(target_hbm.at[pl.ds(base, _BPW)], tidx_v)
    pltpu.sync_copy(context_hbm.at[pl.ds(base, _BPW)], cidx_v)

    cps = []
    for k in range(_NCH):
        sl = pl.ds(k * _CH, _CH)
        cps.append(pltpu.async_copy(emb_hbm.at[tidx_v.at[sl]], trows_v.at[sl], sem))
        cps.append(pltpu.async_copy(outtab_hbm.at[cidx_v.at[sl]], crows_v.at[sl], sem))
    for c in cps:
        c.wait()

    lane = lax.iota(jnp.int32, _L)

    def group(g, carry):
        row = g * _L + lane
        acc = jnp.zeros((_L,), jnp.float32)
        for d in range(DIM):
            col = jnp.full((_L,), d, jnp.int32)
            tv = plsc.load_gather(trows_v, [row, col])
            cv = plsc.load_gather(crows_v, [row, col])
            acc = acc + tv * cv
        res_v[pl.ds(g * _L, _L)] = acc
        return carry

    lax.fori_loop(0, _BPW // _L, group, 0)
    pltpu.sync_copy(res_v, out_hbm.at[pl.ds(base, _BPW)])


def kernel(target, context, embeddings, output):
    mesh = plsc.VectorSubcoreMesh(core_axis_name="c", subcore_axis_name="s")
    f = pl.kernel(
        _body,
        out_type=jax.ShapeDtypeStruct((BATCH,), jnp.float32),
        mesh=mesh,
        scratch_types=[
            pltpu.VMEM((_BPW,), jnp.int32),
            pltpu.VMEM((_BPW,), jnp.int32),
            pltpu.VMEM((_BPW, DIM), jnp.float32),
            pltpu.VMEM((_BPW, DIM), jnp.float32),
            pltpu.VMEM((_BPW,), jnp.float32),
            pltpu.SemaphoreType.DMA,
        ],
        compiler_params=pltpu.CompilerParams(
            needs_layout_passes=False, use_tc_tiling_on_sc=False),
    )
    return f(target.astype(jnp.int32), context.astype(jnp.int32),
             embeddings, output)


# speedup vs baseline: 1.5580x; 1.5580x over previous
import jax
import jax.numpy as jnp
from jax import lax
from jax.experimental import pallas as pl
from jax.experimental.pallas import tpu as pltpu, tpu_sc as plsc

DIM = 64
VOCAB = 1000000
BATCH = 16384

_info = plsc.get_sparse_core_info()
_NC, _NS, _L = _info.num_cores, _info.num_subcores, _info.num_lanes
_NW = _NC * _NS
_BPW = BATCH // _NW
_HALF = _BPW // 2


def _body(target_hbm, context_hbm, emb_hbm, outtab_hbm, out_hbm,
          tidx_s, cidx_s, trows_v, crows_v, res_v, sem):
    wid = lax.axis_index("s") * _NC + lax.axis_index("c")
    base = wid * _BPW

    pltpu.sync_copy(target_hbm.at[pl.ds(base, _BPW)], tidx_s)
    pltpu.sync_copy(context_hbm.at[pl.ds(base, _BPW)], cidx_s)

    lane = lax.iota(jnp.int32, _L)

    for h in range(2):
        off = h * _HALF

        def fire(b, carry):
            tvec = tidx_s[pl.ds(off + b * _L, _L)]
            cvec = cidx_s[pl.ds(off + b * _L, _L)]
            for l in range(_L):
                it = tvec[l]
                ic = cvec[l]
                j = b * _L + l
                pltpu.async_copy(emb_hbm.at[pl.ds(it, 1), :],
                                 trows_v.at[pl.ds(j, 1), :], sem)
                pltpu.async_copy(outtab_hbm.at[pl.ds(ic, 1), :],
                                 crows_v.at[pl.ds(j, 1), :], sem)
            return carry

        lax.fori_loop(0, _HALF // _L, fire, 0)

        def drain(j, carry):
            pltpu.make_async_copy(emb_hbm.at[pl.ds(0, 1), :],
                                  trows_v.at[pl.ds(0, 1), :], sem).wait()
            pltpu.make_async_copy(outtab_hbm.at[pl.ds(0, 1), :],
                                  crows_v.at[pl.ds(0, 1), :], sem).wait()
            return carry

        lax.fori_loop(0, _HALF, drain, 0)

        def group(g, carry):
            row = g * _L + lane
            acc = jnp.zeros((_L,), jnp.float32)
            for d in range(DIM):
                col = jnp.full((_L,), d, jnp.int32)
                tv = plsc.load_gather(trows_v, [row, col])
                cv = plsc.load_gather(crows_v, [row, col])
                acc = acc + tv * cv
            res_v[pl.ds(off + g * _L, _L)] = acc
            return carry

        lax.fori_loop(0, _HALF // _L, group, 0)

    pltpu.sync_copy(res_v, out_hbm.at[pl.ds(base, _BPW)])


def kernel(target, context, embeddings, output):
    mesh = plsc.VectorSubcoreMesh(core_axis_name="c", subcore_axis_name="s")
    f = pl.kernel(
        _body,
        out_type=jax.ShapeDtypeStruct((BATCH,), jnp.float32),
        mesh=mesh,
        scratch_types=[
            pltpu.VMEM((_BPW,), jnp.int32),
            pltpu.VMEM((_BPW,), jnp.int32),
            pltpu.VMEM((_HALF, DIM), jnp.float32),
            pltpu.VMEM((_HALF, DIM), jnp.float32),
            pltpu.VMEM((_BPW,), jnp.float32),
            pltpu.SemaphoreType.DMA,
        ],
        compiler_params=pltpu.CompilerParams(needs_layout_passes=False),
    )
    return f(target.astype(jnp.int32), context.astype(jnp.int32),
             embeddings, output)
